# Initial kernel scaffold; baseline (speedup 1.0000x reference)
import jax
import jax.numpy as jnp
from jax.experimental import pallas as pl


def kernel(features, coords, temb, W1, b1, g1, be1, W2, b2, g2, be2):
    raise NotImplementedError("write your pallas kernel here")


# speedup vs baseline: 1.0016x; 1.0016x over previous
import jax
import jax.numpy as jnp
from jax.experimental import pallas as pl

_M = 1024
_R = 0.2
_K = 32


def _fps(coords, num_centers):
    pts = jnp.transpose(coords, (0, 2, 1))
    B, N, _ = pts.shape
    dists0 = jnp.full((B, N), 1e10, dtype=pts.dtype)
    last0 = jnp.zeros((B,), dtype=jnp.int32)

    def step(carry, _):
        dists, last = carry
        last_pt = jnp.take_along_axis(pts, last[:, None, None], axis=1)
        d = jnp.sum((pts - last_pt) ** 2, axis=-1)
        dists = jnp.minimum(dists, d)
        nxt = jnp.argmax(dists, axis=-1).astype(jnp.int32)
        return (dists, nxt), nxt

    (_, _), rest = jax.lax.scan(step, (dists0, last0), None,
                                length=num_centers - 1)
    idxs = jnp.concatenate([jnp.zeros((1, B), jnp.int32), rest], axis=0)
    return jnp.transpose(idxs, (1, 0))


def _ball_query(centers_coords, points_coords, radius, K):
    c = jnp.transpose(centers_coords, (0, 2, 1))
    p = jnp.transpose(points_coords, (0, 2, 1))
    N = p.shape[1]
    cn = jnp.sum(c * c, axis=-1)
    pn = jnp.sum(p * p, axis=-1)
    dist2 = (cn[:, :, None] + pn[:, None, :]
             - 2.0 * jnp.einsum('bmd,bnd->bmn', c, p))
    within = dist2 < radius * radius
    key = jnp.where(within, jnp.arange(N, dtype=jnp.int32)[None, None, :], N)
    negvals, _ = jax.lax.top_k(-key, K)
    sorted_idx = -negvals
    first = sorted_idx[:, :, :1]
    first = jnp.where(first == N, 0, first)
    return jnp.where(sorted_idx == N, first, sorted_idx).astype(jnp.int32)


def _gn_swish(h, groups, gamma, beta, eps=1e-5):
    C, L = h.shape
    cg = C // groups
    denom = float(cg * L)
    s = jnp.sum(h, axis=1, keepdims=True)
    ss = jnp.sum(h * h, axis=1, keepdims=True)
    gi = jax.lax.broadcasted_iota(jnp.int32, (C, C), 0) // cg
    gj = jax.lax.broadcasted_iota(jnp.int32, (C, C), 1) // cg
    avg = jnp.where(gi == gj, 1.0 / denom, 0.0)
    mean = jnp.dot(avg, s, preferred_element_type=jnp.float32)
    ex2 = jnp.dot(avg, ss, preferred_element_type=jnp.float32)
    var = ex2 - mean * mean
    hn = (h - mean) * jax.lax.rsqrt(var + eps)
    hn = hn * gamma + beta
    return hn * jax.nn.sigmoid(hn)


def _mlp_kernel(x_ref, W1_ref, b1_ref, g1_ref, be1_ref,
                W2_ref, b2_ref, g2_ref, be2_ref, of_ref):
    x = x_ref[0]
    h = jnp.dot(W1_ref[...], x, preferred_element_type=jnp.float32)
    h = h + b1_ref[...]
    h = _gn_swish(h, 8, g1_ref[...], be1_ref[...])
    h = jnp.dot(W2_ref[...], h, preferred_element_type=jnp.float32)
    h = h + b2_ref[...]
    h = _gn_swish(h, 8, g2_ref[...], be2_ref[...])
    of = h[:, 0:_M]
    for k in range(1, _K):
        of = jnp.maximum(of, h[:, k * _M:(k + 1) * _M])
    of_ref[0] = of


def _tmax_kernel(t_ref, o_ref):
    k = pl.program_id(1)
    cur = t_ref[0]

    @pl.when(k == 0)
    def _init():
        o_ref[0] = cur

    @pl.when(k > 0)
    def _acc():
        o_ref[0] = jnp.maximum(o_ref[0], cur)


def _run_mlp(xg, W1, b1, g1, be1, W2, b2, g2, be2):
    B = xg.shape[0]
    KM = _K * _M

    def bmap(b):
        return (b, 0, 0)

    def wmap(b):
        return (0, 0)

    return pl.pallas_call(
        _mlp_kernel,
        grid=(B,),
        in_specs=[
            pl.BlockSpec((1, 35, KM), bmap),
            pl.BlockSpec((32, 35), wmap),
            pl.BlockSpec((32, 1), wmap),
            pl.BlockSpec((32, 1), wmap),
            pl.BlockSpec((32, 1), wmap),
            pl.BlockSpec((64, 32), wmap),
            pl.BlockSpec((64, 1), wmap),
            pl.BlockSpec((64, 1), wmap),
            pl.BlockSpec((64, 1), wmap),
        ],
        out_specs=pl.BlockSpec((1, 64, _M), bmap),
        out_shape=jax.ShapeDtypeStruct((B, 64, _M), jnp.float32),
    )(xg, W1, b1, g1, be1, W2, b2, g2, be2)


def _run_tmax(tg):
    B = tg.shape[0]
    return pl.pallas_call(
        _tmax_kernel,
        grid=(B, _K),
        in_specs=[pl.BlockSpec((1, 64, _M), lambda b, k: (b, 0, k))],
        out_specs=pl.BlockSpec((1, 64, _M), lambda b, k: (b, 0, 0)),
        out_shape=jax.ShapeDtypeStruct((B, 64, _M), jnp.float32),
    )(tg)


@jax.jit
def _forward(features, coords, temb, W1, b1, g1, be1, W2, b2, g2, be2):
    B = features.shape[0]
    pts = jnp.transpose(coords, (0, 2, 1))
    idxs = _fps(jax.lax.stop_gradient(coords), _M)
    centers = jnp.transpose(
        jnp.take_along_axis(pts, idxs[..., None], axis=1), (0, 2, 1))
    nidx = _ball_query(jax.lax.stop_gradient(centers),
                       jax.lax.stop_gradient(coords), _R, _K)
    nidxT = jnp.transpose(nidx, (0, 2, 1))

    gather = jax.vmap(lambda fb, ib: fb[:, ib])
    nc = gather(coords, nidxT) - centers[:, :, None, :]
    nf = jnp.concatenate([nc, gather(features, nidxT)], axis=1)
    nt = gather(temb, nidxT)

    xg = nf.reshape(B, 35, _K * _M)
    col = lambda v: v.reshape(-1, 1)
    out_feat = _run_mlp(xg, W1, col(b1), col(g1), col(be1),
                        W2, col(b2), col(g2), col(be2))
    out_temb = _run_tmax(nt.reshape(B, 64, _K * _M))
    return (out_feat, centers, out_temb)


def kernel(features, coords, temb, W1, b1, g1, be1, W2, b2, g2, be2):
    return _forward(features, coords, temb, W1, b1, g1, be1,
                    W2, b2, g2, be2)
